# Initial kernel scaffold; baseline (speedup 1.0000x reference)
import jax
import jax.numpy as jnp
from jax.experimental import pallas as pl


def kernel(x, patch_w, patch_b, cls_token, pos_embed, sel_w, sel_b, ln1_g, ln1_b, qkv_w, qkv_b, proj_w, proj_b, ln2_g, ln2_b, fc1_w, fc1_b, fc2_w, fc2_b, norm_g, norm_b, head_w, head_b):
    raise NotImplementedError("write your pallas kernel here")


# speedup vs baseline: 1.9114x; 1.9114x over previous
import jax
import jax.numpy as jnp
from jax.experimental import pallas as pl
from jax.experimental.pallas import tpu as pltpu

B, C, IMG, P = 64, 3, 224, 16
D, H, L, DFF, NCLS = 768, 12, 12, 3072, 1000
NPATCH = (IMG // P) ** 2
N = NPATCH + 1
S = 224
DH = D // H
SCALE = DH ** -0.5
LN_EPS = 1e-6
NEG = -1e9

EB = 8
AB = 2

_VMEM_LIMIT = 56 * 1024 * 1024


def _ln_rows(z, g, b):
    m = z.mean(-1, keepdims=True)
    v = ((z - m) ** 2).mean(-1, keepdims=True)
    return (z - m) * jax.lax.rsqrt(v + LN_EPS) * g + b


def _embed_kernel(xp_ref, pw_ref, pb_ref, cls_ref, pos_ref, tok_ref):
    xp = xp_ref[...].reshape(EB * NPATCH, C * P * P)
    patches = jnp.dot(xp, pw_ref[...], preferred_element_type=jnp.float32)
    patches = patches + pb_ref[...]
    patches = patches.reshape(EB, NPATCH, D) + pos_ref[0, 1:, :][None]
    tok_ref[:, 0:1, :] = jnp.broadcast_to(
        (cls_ref[0] + pos_ref[0, 0:1, :])[None], (EB, 1, D))
    tok_ref[:, 1:N, :] = patches
    tok_ref[:, N:, :] = jnp.zeros((EB, S - N, D), jnp.float32)


def _attn_kernel(tok_ref, bias_ref, g1_ref, b1_ref, wqkv_ref, bqkv_ref,
                 wproj_ref, bproj_ref, out_ref, o_scratch):
    t = tok_ref[...].reshape(AB * S, D)
    h = _ln_rows(t, g1_ref[...], b1_ref[...])
    qkv = jnp.dot(h, wqkv_ref[...], preferred_element_type=jnp.float32)
    qkv = qkv + bqkv_ref[...]
    for s in range(AB):
        base = s * S
        brow = bias_ref[s, 0, :][None, :]
        for hh in range(H):
            q = qkv[base:base + S, hh * DH:(hh + 1) * DH]
            k = qkv[base:base + S, D + hh * DH:D + (hh + 1) * DH]
            v = qkv[base:base + S, 2 * D + hh * DH:2 * D + (hh + 1) * DH]
            sc = jax.lax.dot_general(
                q, k, (((1,), (1,)), ((), ())),
                preferred_element_type=jnp.float32) * SCALE + brow
            m = jnp.max(sc, axis=-1, keepdims=True)
            e = jnp.exp(sc - m)
            a = e / jnp.sum(e, axis=-1, keepdims=True)
            o_scratch[base:base + S, hh * DH:(hh + 1) * DH] = jnp.dot(
                a, v, preferred_element_type=jnp.float32)
    proj = jnp.dot(o_scratch[...], wproj_ref[...],
                   preferred_element_type=jnp.float32) + bproj_ref[...]
    out_ref[...] = (t + proj).reshape(AB, S, D)


def _mlp_kernel(tok_ref, g2_ref, b2_ref, w1_ref, b1_ref, w2_ref, b2b_ref,
                out_ref):
    t = tok_ref[...].reshape(AB * S, D)
    h = _ln_rows(t, g2_ref[...], b2_ref[...])
    a1 = jnp.dot(h, w1_ref[...], preferred_element_type=jnp.float32)
    a1 = a1 + b1_ref[...]
    a1 = 0.5 * a1 * (1.0 + jax.lax.erf(a1 * (2.0 ** -0.5)))
    o = jnp.dot(a1, w2_ref[...], preferred_element_type=jnp.float32)
    out_ref[...] = (t + o + b2b_ref[...]).reshape(AB, S, D)


def _head_kernel(cls_ref, g_ref, b_ref, hw_ref, hb_ref, out_ref):
    h = _ln_rows(cls_ref[...], g_ref[...], b_ref[...])
    out_ref[...] = jnp.dot(h, hw_ref[...],
                           preferred_element_type=jnp.float32) + hb_ref[...]


def _full_spec(shape):
    nd = len(shape)
    return pl.BlockSpec(shape, lambda i: (0,) * nd)


def kernel(x, patch_w, patch_b, cls_token, pos_embed, sel_w, sel_b,
           ln1_g, ln1_b, qkv_w, qkv_b, proj_w, proj_b,
           ln2_g, ln2_b, fc1_w, fc1_b, fc2_w, fc2_b,
           norm_g, norm_b, head_w, head_b):
    xp = x.reshape(B, C, IMG // P, P, IMG // P, P).transpose(0, 2, 4, 1, 3, 5)
    xp = xp.reshape(B, NPATCH, C * P * P)
    pw_t = patch_w.T

    tok = pl.pallas_call(
        _embed_kernel,
        grid=(B // EB,),
        in_specs=[
            pl.BlockSpec((EB, NPATCH, C * P * P), lambda i: (i, 0, 0)),
            _full_spec((C * P * P, D)),
            _full_spec((1, D)),
            _full_spec((1, 1, D)),
            _full_spec((1, N, D)),
        ],
        out_specs=pl.BlockSpec((EB, S, D), lambda i: (i, 0, 0)),
        out_shape=jax.ShapeDtypeStruct((B, S, D), jnp.float32),
        compiler_params=pltpu.CompilerParams(
            dimension_semantics=("parallel",),
            vmem_limit_bytes=_VMEM_LIMIT),
        name="vit_embed",
    )(xp, pw_t, patch_b.reshape(1, D), cls_token, pos_embed)

    patches_sel = xp @ patch_w.T + patch_b
    patch_only = jnp.concatenate(
        [jnp.broadcast_to(cls_token, (B, 1, D)), patches_sel],
        axis=1) + pos_embed
    patch_only = patch_only[:, 1:]
    mean_patch = patch_only.mean(axis=1, keepdims=True)
    pm = jnp.concatenate(
        [patch_only, jnp.broadcast_to(mean_patch, (B, NPATCH, D))], axis=-1)
    sel_logits = (pm @ sel_w).squeeze(-1) + sel_b
    keep = jax.nn.sigmoid(sel_logits) > 0.5
    full_mask = jnp.concatenate(
        [jnp.ones((B, 1), dtype=bool), keep,
         jnp.zeros((B, S - N), dtype=bool)], axis=1)
    bias = jnp.where(full_mask, 0.0, NEG).astype(jnp.float32).reshape(B, 1, S)

    attn_call = pl.pallas_call(
        _attn_kernel,
        grid=(B // AB,),
        in_specs=[
            pl.BlockSpec((AB, S, D), lambda i: (i, 0, 0)),
            pl.BlockSpec((AB, 1, S), lambda i: (i, 0, 0)),
            _full_spec((1, D)),
            _full_spec((1, D)),
            _full_spec((D, 3 * D)),
            _full_spec((1, 3 * D)),
            _full_spec((D, D)),
            _full_spec((1, D)),
        ],
        out_specs=pl.BlockSpec((AB, S, D), lambda i: (i, 0, 0)),
        out_shape=jax.ShapeDtypeStruct((B, S, D), jnp.float32),
        scratch_shapes=[pltpu.VMEM((AB * S, D), jnp.float32)],
        compiler_params=pltpu.CompilerParams(
            dimension_semantics=("parallel",),
            vmem_limit_bytes=_VMEM_LIMIT),
        name="vit_attn",
    )

    mlp_call = pl.pallas_call(
        _mlp_kernel,
        grid=(B // AB,),
        in_specs=[
            pl.BlockSpec((AB, S, D), lambda i: (i, 0, 0)),
            _full_spec((1, D)),
            _full_spec((1, D)),
            _full_spec((D, DFF)),
            _full_spec((1, DFF)),
            _full_spec((DFF, D)),
            _full_spec((1, D)),
        ],
        out_specs=pl.BlockSpec((AB, S, D), lambda i: (i, 0, 0)),
        out_shape=jax.ShapeDtypeStruct((B, S, D), jnp.float32),
        compiler_params=pltpu.CompilerParams(
            dimension_semantics=("parallel",),
            vmem_limit_bytes=_VMEM_LIMIT),
        name="vit_mlp",
    )

    def step(t, p):
        g1, b1, qw, qb, pw, pb, g2, b2, f1w, f1b, f2w, f2b = p
        t = attn_call(t, bias, g1.reshape(1, D), b1.reshape(1, D), qw,
                      qb.reshape(1, 3 * D), pw, pb.reshape(1, D))
        t = mlp_call(t, g2.reshape(1, D), b2.reshape(1, D), f1w,
                     f1b.reshape(1, DFF), f2w, f2b.reshape(1, D))
        return t, None

    xs = (ln1_g, ln1_b, qkv_w, qkv_b, proj_w, proj_b,
          ln2_g, ln2_b, fc1_w, fc1_b, fc2_w, fc2_b)
    tok, _ = jax.lax.scan(step, tok, xs)

    logits = pl.pallas_call(
        _head_kernel,
        grid=(1,),
        in_specs=[
            _full_spec((B, D)),
            _full_spec((1, D)),
            _full_spec((1, D)),
            _full_spec((D, NCLS)),
            _full_spec((1, NCLS)),
        ],
        out_specs=_full_spec((B, NCLS)),
        out_shape=jax.ShapeDtypeStruct((B, NCLS), jnp.float32),
        compiler_params=pltpu.CompilerParams(
            dimension_semantics=("arbitrary",),
            vmem_limit_bytes=_VMEM_LIMIT),
        name="vit_head",
    )(tok[:, 0, :], norm_g.reshape(1, D), norm_b.reshape(1, D),
      head_w, head_b.reshape(1, NCLS))
    return logits


# speedup vs baseline: 2.7818x; 1.4553x over previous
import jax
import jax.numpy as jnp
import numpy as np
from jax.experimental import pallas as pl
from jax.experimental.pallas import tpu as pltpu
from jax.sharding import Mesh, PartitionSpec as PS

B, C, IMG, P = 64, 3, 224, 16
D, H, L, DFF, NCLS = 768, 12, 12, 3072, 1000
NPATCH = (IMG // P) ** 2
N = NPATCH + 1
S = 224
DH = D // H
SCALE = DH ** -0.5
LN_EPS = 1e-6
NEG = -1e9

EB = 8
AB = 2

_VMEM_LIMIT = 56 * 1024 * 1024


def _ln_rows(z, g, b):
    m = z.mean(-1, keepdims=True)
    v = ((z - m) ** 2).mean(-1, keepdims=True)
    return (z - m) * jax.lax.rsqrt(v + LN_EPS) * g + b


def _embed_kernel(xp_ref, pw_ref, pb_ref, cls_ref, pos_ref, tok_ref):
    xp = xp_ref[...].reshape(EB * NPATCH, C * P * P)
    patches = jnp.dot(xp, pw_ref[...], preferred_element_type=jnp.float32)
    patches = patches + pb_ref[...]
    patches = patches.reshape(EB, NPATCH, D) + pos_ref[0, 1:, :][None]
    tok_ref[:, 0:1, :] = jnp.broadcast_to(
        (cls_ref[0] + pos_ref[0, 0:1, :])[None], (EB, 1, D))
    tok_ref[:, 1:N, :] = patches
    tok_ref[:, N:, :] = jnp.zeros((EB, S - N, D), jnp.float32)


def _attn_kernel(tok_ref, bias_ref, g1_ref, b1_ref, wqkv_ref, bqkv_ref,
                 wproj_ref, bproj_ref, out_ref, o_scratch):
    t = tok_ref[...].reshape(AB * S, D)
    h = _ln_rows(t, g1_ref[...], b1_ref[...])
    qkv = jnp.dot(h, wqkv_ref[...], preferred_element_type=jnp.float32)
    qkv = qkv + bqkv_ref[...]
    for s in range(AB):
        base = s * S
        brow = bias_ref[s, 0, :][None, :]
        for hh in range(H):
            q = qkv[base:base + S, hh * DH:(hh + 1) * DH]
            k = qkv[base:base + S, D + hh * DH:D + (hh + 1) * DH]
            v = qkv[base:base + S, 2 * D + hh * DH:2 * D + (hh + 1) * DH]
            sc = jax.lax.dot_general(
                q, k, (((1,), (1,)), ((), ())),
                preferred_element_type=jnp.float32) * SCALE + brow
            m = jnp.max(sc, axis=-1, keepdims=True)
            e = jnp.exp(sc - m)
            a = e / jnp.sum(e, axis=-1, keepdims=True)
            o_scratch[base:base + S, hh * DH:(hh + 1) * DH] = jnp.dot(
                a, v, preferred_element_type=jnp.float32)
    proj = jnp.dot(o_scratch[...], wproj_ref[...],
                   preferred_element_type=jnp.float32) + bproj_ref[...]
    out_ref[...] = (t + proj).reshape(AB, S, D)


def _mlp_kernel(tok_ref, g2_ref, b2_ref, w1_ref, b1_ref, w2_ref, b2b_ref,
                out_ref):
    t = tok_ref[...].reshape(AB * S, D)
    h = _ln_rows(t, g2_ref[...], b2_ref[...])
    a1 = jnp.dot(h, w1_ref[...], preferred_element_type=jnp.float32)
    a1 = a1 + b1_ref[...]
    a1 = 0.5 * a1 * (1.0 + jax.lax.erf(a1 * (2.0 ** -0.5)))
    o = jnp.dot(a1, w2_ref[...], preferred_element_type=jnp.float32)
    out_ref[...] = (t + o + b2b_ref[...]).reshape(AB, S, D)


def _head_kernel(cls_ref, g_ref, b_ref, hw_ref, hb_ref, out_ref):
    h = _ln_rows(cls_ref[...], g_ref[...], b_ref[...])
    out_ref[...] = jnp.dot(h, hw_ref[...],
                           preferred_element_type=jnp.float32) + hb_ref[...]


def _full_spec(shape):
    nd = len(shape)
    return pl.BlockSpec(shape, lambda i: (0,) * nd)


def _forward(xp, patch_w, patch_b, cls_token, pos_embed, sel_w, sel_b,
             ln1_g, ln1_b, qkv_w, qkv_b, proj_w, proj_b,
             ln2_g, ln2_b, fc1_w, fc1_b, fc2_w, fc2_b,
             norm_g, norm_b, head_w, head_b):
    bl = xp.shape[0]
    pw_t = patch_w.T

    tok = pl.pallas_call(
        _embed_kernel,
        grid=(bl // EB,),
        in_specs=[
            pl.BlockSpec((EB, NPATCH, C * P * P), lambda i: (i, 0, 0)),
            _full_spec((C * P * P, D)),
            _full_spec((1, D)),
            _full_spec((1, 1, D)),
            _full_spec((1, N, D)),
        ],
        out_specs=pl.BlockSpec((EB, S, D), lambda i: (i, 0, 0)),
        out_shape=jax.ShapeDtypeStruct((bl, S, D), jnp.float32),
        compiler_params=pltpu.CompilerParams(
            dimension_semantics=("parallel",),
            vmem_limit_bytes=_VMEM_LIMIT),
        name="vit_embed",
    )(xp, pw_t, patch_b.reshape(1, D), cls_token, pos_embed)

    patches_sel = xp @ patch_w.T + patch_b
    patch_only = jnp.concatenate(
        [jnp.broadcast_to(cls_token, (bl, 1, D)), patches_sel],
        axis=1) + pos_embed
    patch_only = patch_only[:, 1:]
    mean_patch = patch_only.mean(axis=1, keepdims=True)
    pm = jnp.concatenate(
        [patch_only, jnp.broadcast_to(mean_patch, (bl, NPATCH, D))], axis=-1)
    sel_logits = (pm @ sel_w).squeeze(-1) + sel_b
    keep = jax.nn.sigmoid(sel_logits) > 0.5
    full_mask = jnp.concatenate(
        [jnp.ones((bl, 1), dtype=bool), keep,
         jnp.zeros((bl, S - N), dtype=bool)], axis=1)
    bias = jnp.where(full_mask, 0.0, NEG).astype(jnp.float32).reshape(bl, 1, S)

    attn_call = pl.pallas_call(
        _attn_kernel,
        grid=(bl // AB,),
        in_specs=[
            pl.BlockSpec((AB, S, D), lambda i: (i, 0, 0)),
            pl.BlockSpec((AB, 1, S), lambda i: (i, 0, 0)),
            _full_spec((1, D)),
            _full_spec((1, D)),
            _full_spec((D, 3 * D)),
            _full_spec((1, 3 * D)),
            _full_spec((D, D)),
            _full_spec((1, D)),
        ],
        out_specs=pl.BlockSpec((AB, S, D), lambda i: (i, 0, 0)),
        out_shape=jax.ShapeDtypeStruct((bl, S, D), jnp.float32),
        scratch_shapes=[pltpu.VMEM((AB * S, D), jnp.float32)],
        compiler_params=pltpu.CompilerParams(
            dimension_semantics=("parallel",),
            vmem_limit_bytes=_VMEM_LIMIT),
        name="vit_attn",
    )

    mlp_call = pl.pallas_call(
        _mlp_kernel,
        grid=(bl // AB,),
        in_specs=[
            pl.BlockSpec((AB, S, D), lambda i: (i, 0, 0)),
            _full_spec((1, D)),
            _full_spec((1, D)),
            _full_spec((D, DFF)),
            _full_spec((1, DFF)),
            _full_spec((DFF, D)),
            _full_spec((1, D)),
        ],
        out_specs=pl.BlockSpec((AB, S, D), lambda i: (i, 0, 0)),
        out_shape=jax.ShapeDtypeStruct((bl, S, D), jnp.float32),
        compiler_params=pltpu.CompilerParams(
            dimension_semantics=("parallel",),
            vmem_limit_bytes=_VMEM_LIMIT),
        name="vit_mlp",
    )

    def step(t, p):
        g1, b1, qw, qb, pw, pb, g2, b2, f1w, f1b, f2w, f2b = p
        t = attn_call(t, bias, g1.reshape(1, D), b1.reshape(1, D), qw,
                      qb.reshape(1, 3 * D), pw, pb.reshape(1, D))
        t = mlp_call(t, g2.reshape(1, D), b2.reshape(1, D), f1w,
                     f1b.reshape(1, DFF), f2w, f2b.reshape(1, D))
        return t, None

    xs = (ln1_g, ln1_b, qkv_w, qkv_b, proj_w, proj_b,
          ln2_g, ln2_b, fc1_w, fc1_b, fc2_w, fc2_b)
    tok, _ = jax.lax.scan(step, tok, xs)

    logits = pl.pallas_call(
        _head_kernel,
        grid=(1,),
        in_specs=[
            _full_spec((bl, D)),
            _full_spec((1, D)),
            _full_spec((1, D)),
            _full_spec((D, NCLS)),
            _full_spec((1, NCLS)),
        ],
        out_specs=_full_spec((bl, NCLS)),
        out_shape=jax.ShapeDtypeStruct((bl, NCLS), jnp.float32),
        compiler_params=pltpu.CompilerParams(
            dimension_semantics=("arbitrary",),
            vmem_limit_bytes=_VMEM_LIMIT),
        name="vit_head",
    )(tok[:, 0, :], norm_g.reshape(1, D), norm_b.reshape(1, D),
      head_w, head_b.reshape(1, NCLS))
    return logits


def kernel(x, patch_w, patch_b, cls_token, pos_embed, sel_w, sel_b,
           ln1_g, ln1_b, qkv_w, qkv_b, proj_w, proj_b,
           ln2_g, ln2_b, fc1_w, fc1_b, fc2_w, fc2_b,
           norm_g, norm_b, head_w, head_b):
    xp = x.reshape(B, C, IMG // P, P, IMG // P, P).transpose(0, 2, 4, 1, 3, 5)
    xp = xp.reshape(B, NPATCH, C * P * P)
    rest = (patch_w, patch_b, cls_token, pos_embed, sel_w, sel_b,
            ln1_g, ln1_b, qkv_w, qkv_b, proj_w, proj_b,
            ln2_g, ln2_b, fc1_w, fc1_b, fc2_w, fc2_b,
            norm_g, norm_b, head_w, head_b)

    devs = jax.devices()
    ndev = 2 if (len(devs) >= 2 and B % (2 * EB) == 0) else 1
    if ndev == 1:
        return _forward(xp, *rest)

    mesh = Mesh(np.array(devs[:ndev]), ("b",))
    fwd = jax.shard_map(
        _forward, mesh=mesh,
        in_specs=(PS("b"),) + (PS(),) * len(rest),
        out_specs=PS("b"), check_vma=False)
    return fwd(xp, *rest)


# speedup vs baseline: 2.9465x; 1.0592x over previous
import jax
import jax.numpy as jnp
import numpy as np
from jax.experimental import pallas as pl
from jax.experimental.pallas import tpu as pltpu
from jax.sharding import Mesh, PartitionSpec as PS

B, C, IMG, P = 64, 3, 224, 16
D, H, L, DFF, NCLS = 768, 12, 12, 3072, 1000
NPATCH = (IMG // P) ** 2
N = NPATCH + 1
S = 256
SB = 128
NBLK = S // SB
DH = D // H
SCALE = DH ** -0.5
LN_EPS = 1e-6
NEG = -1e9

EB = 8
AB = 2

_VMEM_LIMIT = 56 * 1024 * 1024


def _ln_rows(z, g, b):
    m = z.mean(-1, keepdims=True)
    v = ((z - m) ** 2).mean(-1, keepdims=True)
    return (z - m) * jax.lax.rsqrt(v + LN_EPS) * g + b


def _embed_kernel(xp_ref, pw_ref, pb_ref, cls_ref, pos_ref, perm_ref, tok_ref):
    xp = xp_ref[...].reshape(EB * NPATCH, C * P * P)
    patches = jnp.dot(xp, pw_ref[...], preferred_element_type=jnp.float32)
    patches = patches + pb_ref[...]
    patches = patches.reshape(EB, NPATCH, D) + pos_ref[0, 1:, :][None]
    cls_row = cls_ref[0] + pos_ref[0, 0:1, :]
    row_iota = jax.lax.broadcasted_iota(jnp.int32, (N, S), 0)
    for s in range(EB):
        tokd = jnp.concatenate([cls_row, patches[s]], axis=0)
        onehot_t = jnp.where(row_iota == perm_ref[s, 0, :][None, :],
                             1.0, 0.0)
        tok_ref[s] = jax.lax.dot_general(
            onehot_t, tokd, (((0,), (0,)), ((), ())),
            preferred_element_type=jnp.float32)


def _attn_kernel(nb_ref, tok_ref, bias_ref, g1_ref, b1_ref, wqkv_ref,
                 bqkv_ref, wproj_ref, bproj_ref, out_ref, qkv_scr, o_scr):
    step = pl.program_id(0)
    for s in range(AB):
        nb = nb_ref[step * AB + s]
        for j in range(NBLK):
            if j == 0:
                t = tok_ref[s, :SB, :]
                h = _ln_rows(t, g1_ref[...], b1_ref[...])
                qkv_scr[s, :SB, :] = jnp.dot(
                    h, wqkv_ref[...],
                    preferred_element_type=jnp.float32) + bqkv_ref[...]
            else:
                @pl.when(j < nb)
                def _(s=s, j=j):
                    t = tok_ref[s, j * SB:(j + 1) * SB, :]
                    h = _ln_rows(t, g1_ref[...], b1_ref[...])
                    qkv_scr[s, j * SB:(j + 1) * SB, :] = jnp.dot(
                        h, wqkv_ref[...],
                        preferred_element_type=jnp.float32) + bqkv_ref[...]

                @pl.when(j >= nb)
                def _(s=s, j=j):
                    qkv_scr[s, j * SB:(j + 1) * SB, :] = jnp.zeros(
                        (SB, 3 * D), jnp.float32)
        brow = bias_ref[s, 0, :][None, :]
        for i in range(NBLK):
            def _attn_block(s=s, i=i, brow=brow):
                rows = pl.ds(i * SB, SB)
                for hh in range(H):
                    q = qkv_scr[s, rows, hh * DH:(hh + 1) * DH]
                    k = qkv_scr[s, :, D + hh * DH:D + (hh + 1) * DH]
                    v = qkv_scr[s, :, 2 * D + hh * DH:2 * D + (hh + 1) * DH]
                    sc = jax.lax.dot_general(
                        q, k, (((1,), (1,)), ((), ())),
                        preferred_element_type=jnp.float32) * SCALE + brow
                    m = jnp.max(sc, axis=-1, keepdims=True)
                    e = jnp.exp(sc - m)
                    a = e / jnp.sum(e, axis=-1, keepdims=True)
                    o_scr[s, rows, hh * DH:(hh + 1) * DH] = jnp.dot(
                        a, v, preferred_element_type=jnp.float32)
                proj = jnp.dot(o_scr[s, rows, :], wproj_ref[...],
                               preferred_element_type=jnp.float32)
                out_ref[s, rows, :] = (tok_ref[s, rows, :] + proj
                                       + bproj_ref[...])
            if i == 0:
                _attn_block()
            else:
                pl.when(i < nb)(_attn_block)


def _mlp_kernel(nb_ref, tok_ref, g2_ref, b2_ref, w1_ref, b1_ref, w2_ref,
                b2b_ref, out_ref):
    step = pl.program_id(0)
    for s in range(AB):
        nb = nb_ref[step * AB + s]
        for j in range(NBLK):
            def _mlp_block(s=s, j=j):
                rows = pl.ds(j * SB, SB)
                t = tok_ref[s, rows, :]
                h = _ln_rows(t, g2_ref[...], b2_ref[...])
                a1 = jnp.dot(h, w1_ref[...],
                             preferred_element_type=jnp.float32) + b1_ref[...]
                a1 = 0.5 * a1 * (1.0 + jax.lax.erf(a1 * (2.0 ** -0.5)))
                o = jnp.dot(a1, w2_ref[...],
                            preferred_element_type=jnp.float32)
                out_ref[s, rows, :] = t + o + b2b_ref[...]
            if j == 0:
                _mlp_block()
            else:
                pl.when(j < nb)(_mlp_block)


def _head_kernel(cls_ref, g_ref, b_ref, hw_ref, hb_ref, out_ref):
    h = _ln_rows(cls_ref[...], g_ref[...], b_ref[...])
    out_ref[...] = jnp.dot(h, hw_ref[...],
                           preferred_element_type=jnp.float32) + hb_ref[...]


def _full_spec(shape, np_extra=0):
    nd = len(shape)
    return pl.BlockSpec(shape, lambda i, *_: (0,) * nd)


def _forward(xp, patch_w, patch_b, cls_token, pos_embed, sel_w, sel_b,
             ln1_g, ln1_b, qkv_w, qkv_b, proj_w, proj_b,
             ln2_g, ln2_b, fc1_w, fc1_b, fc2_w, fc2_b,
             norm_g, norm_b, head_w, head_b):
    bl = xp.shape[0]
    pw_t = patch_w.T

    patches_sel = xp @ patch_w.T + patch_b
    patch_only = jnp.concatenate(
        [jnp.broadcast_to(cls_token, (bl, 1, D)), patches_sel],
        axis=1) + pos_embed
    patch_only = patch_only[:, 1:]
    mean_patch = patch_only.mean(axis=1, keepdims=True)
    pm = jnp.concatenate(
        [patch_only, jnp.broadcast_to(mean_patch, (bl, NPATCH, D))], axis=-1)
    sel_logits = (pm @ sel_w).squeeze(-1) + sel_b
    keep = jax.nn.sigmoid(sel_logits) > 0.5
    full_mask = jnp.concatenate(
        [jnp.ones((bl, 1), dtype=bool), keep], axis=1)

    perm = jnp.argsort(jnp.where(full_mask, 0, 1), axis=1, stable=True)
    perm = jnp.concatenate(
        [perm, jnp.full((bl, S - N), N + 99, jnp.int32)],
        axis=1).astype(jnp.int32).reshape(bl, 1, S)
    counts = jnp.sum(full_mask, axis=1).astype(jnp.int32)
    nb_arr = (counts + (SB - 1)) // SB
    col_iota = jax.lax.broadcasted_iota(jnp.int32, (bl, S), 1)
    bias = jnp.where(col_iota < counts[:, None], 0.0, NEG)
    bias = bias.astype(jnp.float32).reshape(bl, 1, S)

    tok = pl.pallas_call(
        _embed_kernel,
        grid=(bl // EB,),
        in_specs=[
            pl.BlockSpec((EB, NPATCH, C * P * P), lambda i: (i, 0, 0)),
            pl.BlockSpec((C * P * P, D), lambda i: (0, 0)),
            pl.BlockSpec((1, D), lambda i: (0, 0)),
            pl.BlockSpec((1, 1, D), lambda i: (0, 0, 0)),
            pl.BlockSpec((1, N, D), lambda i: (0, 0, 0)),
            pl.BlockSpec((EB, 1, S), lambda i: (i, 0, 0)),
        ],
        out_specs=pl.BlockSpec((EB, S, D), lambda i: (i, 0, 0)),
        out_shape=jax.ShapeDtypeStruct((bl, S, D), jnp.float32),
        compiler_params=pltpu.CompilerParams(
            dimension_semantics=("parallel",),
            vmem_limit_bytes=_VMEM_LIMIT),
        name="vit_embed",
    )(xp, pw_t, patch_b.reshape(1, D), cls_token, pos_embed, perm)

    attn_call = pl.pallas_call(
        _attn_kernel,
        grid_spec=pltpu.PrefetchScalarGridSpec(
            num_scalar_prefetch=1,
            grid=(bl // AB,),
            in_specs=[
                pl.BlockSpec((AB, S, D), lambda i, *_: (i, 0, 0)),
                pl.BlockSpec((AB, 1, S), lambda i, *_: (i, 0, 0)),
                _full_spec((1, D)),
                _full_spec((1, D)),
                _full_spec((D, 3 * D)),
                _full_spec((1, 3 * D)),
                _full_spec((D, D)),
                _full_spec((1, D)),
            ],
            out_specs=pl.BlockSpec((AB, S, D), lambda i, *_: (i, 0, 0)),
            scratch_shapes=[
                pltpu.VMEM((AB, S, 3 * D), jnp.float32),
                pltpu.VMEM((AB, S, D), jnp.float32),
            ]),
        out_shape=jax.ShapeDtypeStruct((bl, S, D), jnp.float32),
        compiler_params=pltpu.CompilerParams(
            dimension_semantics=("parallel",),
            vmem_limit_bytes=_VMEM_LIMIT),
        name="vit_attn",
    )

    mlp_call = pl.pallas_call(
        _mlp_kernel,
        grid_spec=pltpu.PrefetchScalarGridSpec(
            num_scalar_prefetch=1,
            grid=(bl // AB,),
            in_specs=[
                pl.BlockSpec((AB, S, D), lambda i, *_: (i, 0, 0)),
                _full_spec((1, D)),
                _full_spec((1, D)),
                _full_spec((D, DFF)),
                _full_spec((1, DFF)),
                _full_spec((DFF, D)),
                _full_spec((1, D)),
            ],
            out_specs=pl.BlockSpec((AB, S, D), lambda i, *_: (i, 0, 0))),
        out_shape=jax.ShapeDtypeStruct((bl, S, D), jnp.float32),
        compiler_params=pltpu.CompilerParams(
            dimension_semantics=("parallel",),
            vmem_limit_bytes=_VMEM_LIMIT),
        name="vit_mlp",
    )

    def step(t, p):
        g1, b1, qw, qb, pw, pb, g2, b2, f1w, f1b, f2w, f2b = p
        t = attn_call(nb_arr, t, bias, g1.reshape(1, D), b1.reshape(1, D), qw,
                      qb.reshape(1, 3 * D), pw, pb.reshape(1, D))
        t = mlp_call(nb_arr, t, g2.reshape(1, D), b2.reshape(1, D), f1w,
                     f1b.reshape(1, DFF), f2w, f2b.reshape(1, D))
        return t, None

    xs = (ln1_g, ln1_b, qkv_w, qkv_b, proj_w, proj_b,
          ln2_g, ln2_b, fc1_w, fc1_b, fc2_w, fc2_b)
    tok, _ = jax.lax.scan(step, tok, xs)

    logits = pl.pallas_call(
        _head_kernel,
        grid=(1,),
        in_specs=[
            pl.BlockSpec((bl, D), lambda i: (0, 0)),
            pl.BlockSpec((1, D), lambda i: (0, 0)),
            pl.BlockSpec((1, D), lambda i: (0, 0)),
            pl.BlockSpec((D, NCLS), lambda i: (0, 0)),
            pl.BlockSpec((1, NCLS), lambda i: (0, 0)),
        ],
        out_specs=pl.BlockSpec((bl, NCLS), lambda i: (0, 0)),
        out_shape=jax.ShapeDtypeStruct((bl, NCLS), jnp.float32),
        compiler_params=pltpu.CompilerParams(
            dimension_semantics=("arbitrary",),
            vmem_limit_bytes=_VMEM_LIMIT),
        name="vit_head",
    )(tok[:, 0, :], norm_g.reshape(1, D), norm_b.reshape(1, D),
      head_w, head_b.reshape(1, NCLS))
    return logits


def kernel(x, patch_w, patch_b, cls_token, pos_embed, sel_w, sel_b,
           ln1_g, ln1_b, qkv_w, qkv_b, proj_w, proj_b,
           ln2_g, ln2_b, fc1_w, fc1_b, fc2_w, fc2_b,
           norm_g, norm_b, head_w, head_b):
    xp = x.reshape(B, C, IMG // P, P, IMG // P, P).transpose(0, 2, 4, 1, 3, 5)
    xp = xp.reshape(B, NPATCH, C * P * P)
    rest = (patch_w, patch_b, cls_token, pos_embed, sel_w, sel_b,
            ln1_g, ln1_b, qkv_w, qkv_b, proj_w, proj_b,
            ln2_g, ln2_b, fc1_w, fc1_b, fc2_w, fc2_b,
            norm_g, norm_b, head_w, head_b)

    devs = jax.devices()
    ndev = 2 if (len(devs) >= 2 and B % (2 * EB) == 0) else 1
    if ndev == 1:
        return _forward(xp, *rest)

    mesh = Mesh(np.array(devs[:ndev]), ("b",))
    fwd = jax.shard_map(
        _forward, mesh=mesh,
        in_specs=(PS("b"),) + (PS(),) * len(rest),
        out_specs=PS("b"), check_vma=False)
    return fwd(xp, *rest)
